# baseline (speedup 1.0000x reference)
import functools

import jax
import jax.numpy as jnp
from jax import lax
from jax.experimental import pallas as pl
from jax.experimental.pallas import tpu as pltpu
from jax.experimental.pallas import tpu_sc as plsc

_N1, _N2 = 16384, 4096
_BLK = 512
_NW = 32
_CHUNK = 128


def _knn_body(p1_ref, x1_ref, p2t_ref, w1_ref, b_ref,
              h1_ref, i0_ref, i1_ref, i2_ref, w0_ref, w1o_ref, w2o_ref):
    p1b = p1_ref[...]
    p2t = p2t_ref[...]
    dot = jnp.dot(p1b, p2t, preferred_element_type=jnp.float32)
    n1 = jnp.sum(p1b * p1b, axis=1, keepdims=True)
    n2 = jnp.sum(p2t * p2t, axis=0, keepdims=True)
    d2 = (n1 + n2) - 2.0 * dot

    cols = jax.lax.broadcasted_iota(jnp.int32, d2.shape, 1)
    invs, sels = [], []
    for _ in range(3):
        m = jnp.min(d2, axis=1, keepdims=True)
        sel = jnp.min(jnp.where(d2 == m, cols, _N2), axis=1, keepdims=True)
        dist = jnp.sqrt(jnp.maximum(m, 0.0))
        invs.append(1.0 / (dist + 1e-8))
        sels.append(sel)
        d2 = jnp.where(cols == sel, jnp.inf, d2)
    wsum = invs[0] + invs[1] + invs[2]

    i0_ref[...], i1_ref[...], i2_ref[...] = sels
    w0_ref[...] = invs[0] / wsum
    w1o_ref[...] = invs[1] / wsum
    w2o_ref[...] = invs[2] / wsum

    h1_ref[...] = (jnp.dot(x1_ref[...], w1_ref[...],
                           preferred_element_type=jnp.float32) + b_ref[...])


def _sc_gather_body(x2_hbm, i0_hbm, i1_hbm, i2_hbm,
                    g0_hbm, g1_hbm, g2_hbm,
                    iv0, iv1, iv2, buf0, buf1, buf2, sem):
    wid = lax.axis_index("s") * 2 + lax.axis_index("c")
    base = wid * (_N1 // _NW)

    def chunk(c, carry):
        off = base + c * _CHUNK
        sl = pl.ds(off, _CHUNK)
        pltpu.sync_copy(i0_hbm.at[sl], iv0)
        pltpu.sync_copy(i1_hbm.at[sl], iv1)
        pltpu.sync_copy(i2_hbm.at[sl], iv2)
        c0 = pltpu.async_copy(x2_hbm.at[iv0], buf0, sem)
        c1 = pltpu.async_copy(x2_hbm.at[iv1], buf1, sem)
        c2 = pltpu.async_copy(x2_hbm.at[iv2], buf2, sem)
        c0.wait()
        c1.wait()
        c2.wait()
        pltpu.sync_copy(buf0, g0_hbm.at[sl])
        pltpu.sync_copy(buf1, g1_hbm.at[sl])
        pltpu.sync_copy(buf2, g2_hbm.at[sl])
        return carry

    lax.fori_loop(0, (_N1 // _NW) // _CHUNK, chunk, 0)


def _combine_body(h1_ref, g0_ref, g1_ref, g2_ref, w0_ref, w1_ref, w2_ref,
                  wm_ref, h_ref, sum_ref, sumsq_ref):
    i = pl.program_id(0)
    interp = (w0_ref[...] * g0_ref[...] + w1_ref[...] * g1_ref[...]
              + w2_ref[...] * g2_ref[...])
    h = h1_ref[...] + jnp.dot(interp, wm_ref[...],
                              preferred_element_type=jnp.float32)
    h_ref[...] = h

    @pl.when(i == 0)
    def _init():
        sum_ref[...] = jnp.zeros_like(sum_ref)
        sumsq_ref[...] = jnp.zeros_like(sumsq_ref)

    sum_ref[...] += jnp.sum(h, axis=0, keepdims=True)
    sumsq_ref[...] += jnp.sum(h * h, axis=0, keepdims=True)


def _bn_body(h_ref, sum_ref, sumsq_ref, gamma_ref, beta_ref, out_ref):
    inv_n = 1.0 / _N1
    mean = sum_ref[...] * inv_n
    var = sumsq_ref[...] * inv_n - mean * mean
    scale = gamma_ref[...] / jnp.sqrt(var + 1e-5)
    shift = beta_ref[...] - mean * scale
    out_ref[...] = jnp.maximum(h_ref[...] * scale + shift, 0.0)


def kernel(p1, x1, o1, p2, x2, o2, W, b, gamma, beta):
    d_dense = x1.shape[1]
    d_sparse = x2.shape[1]
    d_out = W.shape[1]
    p1p = jnp.pad(p1, ((0, 0), (0, 5)))
    p2t = jnp.pad(p2, ((0, 0), (0, 5))).T
    w1 = W[:d_dense]
    wm = W[d_dense:]
    b2 = b.reshape(1, d_out)

    grid = (_N1 // _BLK,)
    cst = lambda i: (0, 0)
    row = lambda i: (i, 0)
    h1, i0, i1, i2, w0, w1n, w2n = pl.pallas_call(
        _knn_body,
        grid=grid,
        in_specs=[
            pl.BlockSpec((_BLK, 8), row),
            pl.BlockSpec((_BLK, d_dense), row),
            pl.BlockSpec((8, _N2), cst),
            pl.BlockSpec((d_dense, d_out), cst),
            pl.BlockSpec((1, d_out), cst),
        ],
        out_specs=[
            pl.BlockSpec((_BLK, d_out), row),
            pl.BlockSpec((_BLK, 1), row),
            pl.BlockSpec((_BLK, 1), row),
            pl.BlockSpec((_BLK, 1), row),
            pl.BlockSpec((_BLK, 1), row),
            pl.BlockSpec((_BLK, 1), row),
            pl.BlockSpec((_BLK, 1), row),
        ],
        out_shape=[
            jax.ShapeDtypeStruct((_N1, d_out), jnp.float32),
            jax.ShapeDtypeStruct((_N1, 1), jnp.int32),
            jax.ShapeDtypeStruct((_N1, 1), jnp.int32),
            jax.ShapeDtypeStruct((_N1, 1), jnp.int32),
            jax.ShapeDtypeStruct((_N1, 1), jnp.float32),
            jax.ShapeDtypeStruct((_N1, 1), jnp.float32),
            jax.ShapeDtypeStruct((_N1, 1), jnp.float32),
        ],
    )(p1p, x1, p2t, w1, b2)

    sc_gather = functools.partial(
        pl.kernel,
        mesh=plsc.VectorSubcoreMesh(core_axis_name="c", subcore_axis_name="s"),
        out_type=[
            jax.ShapeDtypeStruct((_N1, d_sparse), jnp.float32),
            jax.ShapeDtypeStruct((_N1, d_sparse), jnp.float32),
            jax.ShapeDtypeStruct((_N1, d_sparse), jnp.float32),
        ],
        scratch_types=[
            pltpu.VMEM((_CHUNK,), jnp.int32),
            pltpu.VMEM((_CHUNK,), jnp.int32),
            pltpu.VMEM((_CHUNK,), jnp.int32),
            pltpu.VMEM((_CHUNK, d_sparse), jnp.float32),
            pltpu.VMEM((_CHUNK, d_sparse), jnp.float32),
            pltpu.VMEM((_CHUNK, d_sparse), jnp.float32),
            pltpu.SemaphoreType.DMA,
        ],
    )(_sc_gather_body)
    g0, g1, g2 = sc_gather(x2, i0.reshape(_N1), i1.reshape(_N1),
                           i2.reshape(_N1))

    h, s1, s2 = pl.pallas_call(
        _combine_body,
        grid=grid,
        in_specs=[
            pl.BlockSpec((_BLK, d_out), row),
            pl.BlockSpec((_BLK, d_sparse), row),
            pl.BlockSpec((_BLK, d_sparse), row),
            pl.BlockSpec((_BLK, d_sparse), row),
            pl.BlockSpec((_BLK, 1), row),
            pl.BlockSpec((_BLK, 1), row),
            pl.BlockSpec((_BLK, 1), row),
            pl.BlockSpec((d_sparse, d_out), cst),
        ],
        out_specs=[
            pl.BlockSpec((_BLK, d_out), row),
            pl.BlockSpec((1, d_out), cst),
            pl.BlockSpec((1, d_out), cst),
        ],
        out_shape=[
            jax.ShapeDtypeStruct((_N1, d_out), jnp.float32),
            jax.ShapeDtypeStruct((1, d_out), jnp.float32),
            jax.ShapeDtypeStruct((1, d_out), jnp.float32),
        ],
    )(h1, g0, g1, g2, w0, w1n, w2n, wm)

    x = pl.pallas_call(
        _bn_body,
        grid=grid,
        in_specs=[
            pl.BlockSpec((_BLK, d_out), row),
            pl.BlockSpec((1, d_out), cst),
            pl.BlockSpec((1, d_out), cst),
            pl.BlockSpec((1, d_out), cst),
            pl.BlockSpec((1, d_out), cst),
        ],
        out_specs=pl.BlockSpec((_BLK, d_out), row),
        out_shape=jax.ShapeDtypeStruct((_N1, d_out), jnp.float32),
    )(h, s1, s2, gamma.reshape(1, d_out), beta.reshape(1, d_out))

    return (p1, x, o1)


# speedup vs baseline: 1.1386x; 1.1386x over previous
import functools

import jax
import jax.numpy as jnp
from jax import lax
from jax.experimental import pallas as pl
from jax.experimental.pallas import tpu as pltpu
from jax.experimental.pallas import tpu_sc as plsc

_N1, _N2 = 16384, 4096
_BLK = 512
_NB = _N1 // _BLK
_NW = 32
_CHUNK = 128


def _knn_body(p1e_ref, x1_ref, p2e_ref, w1_ref, b_ref,
              h1_ref, i0_ref, i1_ref, i2_ref, w0_ref, w1o_ref, w2o_ref):
    p1b = p1e_ref[...]
    p2t = p2e_ref[...]
    dot = jnp.dot(p1b, p2t, preferred_element_type=jnp.float32)
    n1 = jnp.sum(p1b * p1b, axis=1, keepdims=True)
    n2 = jnp.sum(p2t * p2t, axis=0, keepdims=True)
    d2 = (n1 + n2) - 2.0 * dot

    colsf = lax.broadcasted_iota(jnp.int32, d2.shape, 1).astype(jnp.float32)
    invs, sels = [], []
    for _ in range(3):
        m = jnp.min(d2, axis=1, keepdims=True)
        sel = jnp.min(jnp.where(d2 == m, colsf, float(_N2)),
                      axis=1, keepdims=True)
        dist = jnp.sqrt(jnp.maximum(m, 0.0))
        invs.append(1.0 / (dist + 1e-8))
        sels.append(sel)
        d2 = jnp.where(colsf == sel, jnp.inf, d2)
    wsum = invs[0] + invs[1] + invs[2]

    i0_ref[...] = sels[0].astype(jnp.int32)
    i1_ref[...] = sels[1].astype(jnp.int32)
    i2_ref[...] = sels[2].astype(jnp.int32)
    w0_ref[...] = invs[0] / wsum
    w1o_ref[...] = invs[1] / wsum
    w2o_ref[...] = invs[2] / wsum

    h1_ref[...] = (jnp.dot(x1_ref[...], w1_ref[...],
                           preferred_element_type=jnp.float32) + b_ref[...])


def _sc_gather_body(x2_hbm, i0_hbm, i1_hbm, i2_hbm,
                    g0_hbm, g1_hbm, g2_hbm,
                    iv0, iv1, iv2, buf0, buf1, buf2, sem):
    wid = lax.axis_index("s") * 2 + lax.axis_index("c")
    base = wid * (_N1 // _NW)

    def chunk(c, carry):
        off = base + c * _CHUNK
        sl = pl.ds(off, _CHUNK)
        pltpu.sync_copy(i0_hbm.at[sl], iv0)
        pltpu.sync_copy(i1_hbm.at[sl], iv1)
        pltpu.sync_copy(i2_hbm.at[sl], iv2)
        c0 = pltpu.async_copy(x2_hbm.at[iv0], buf0, sem)
        c1 = pltpu.async_copy(x2_hbm.at[iv1], buf1, sem)
        c2 = pltpu.async_copy(x2_hbm.at[iv2], buf2, sem)
        c0.wait()
        c1.wait()
        c2.wait()
        pltpu.sync_copy(buf0, g0_hbm.at[sl])
        pltpu.sync_copy(buf1, g1_hbm.at[sl])
        pltpu.sync_copy(buf2, g2_hbm.at[sl])
        return carry

    lax.fori_loop(0, (_N1 // _NW) // _CHUNK, chunk, 0)


def _bn_fused_body(h1_ref, g0_ref, g1_ref, g2_ref, w0_ref, w1_ref, w2_ref,
                   wm_ref, gamma_ref, beta_ref, x_ref,
                   h_scr, sum_scr, sumsq_scr):
    i = pl.program_id(0)

    @pl.when(i == 0)
    def _init():
        sum_scr[...] = jnp.zeros_like(sum_scr)
        sumsq_scr[...] = jnp.zeros_like(sumsq_scr)

    @pl.when(i < _NB)
    def _phase0():
        interp = (w0_ref[...] * g0_ref[...] + w1_ref[...] * g1_ref[...]
                  + w2_ref[...] * g2_ref[...])
        h = h1_ref[...] + jnp.dot(interp, wm_ref[...],
                                  preferred_element_type=jnp.float32)
        h_scr[pl.ds(i * _BLK, _BLK), :] = h
        sum_scr[...] += jnp.sum(h, axis=0, keepdims=True)
        sumsq_scr[...] += jnp.sum(h * h, axis=0, keepdims=True)

    @pl.when(i >= _NB)
    def _phase1():
        j = i - _NB
        inv_n = 1.0 / _N1
        mean = sum_scr[...] * inv_n
        var = sumsq_scr[...] * inv_n - mean * mean
        scale = gamma_ref[...] / jnp.sqrt(var + 1e-5)
        shift = beta_ref[...] - mean * scale
        h = h_scr[pl.ds(j * _BLK, _BLK), :]
        x_ref[...] = jnp.maximum(h * scale + shift, 0.0)


def kernel(p1, x1, o1, p2, x2, o2, W, b, gamma, beta):
    d_dense = x1.shape[1]
    d_sparse = x2.shape[1]
    d_out = W.shape[1]

    p1e = jnp.pad(p1, ((0, 0), (0, 5)))
    p2e = jnp.pad(p2, ((0, 0), (0, 5))).T

    w1 = W[:d_dense]
    wm = W[d_dense:]
    b2 = b.reshape(1, d_out)

    cst = lambda i: (0, 0)
    row = lambda i: (i, 0)
    h1, i0, i1, i2, w0, w1n, w2n = pl.pallas_call(
        _knn_body,
        grid=(_NB,),
        in_specs=[
            pl.BlockSpec((_BLK, 8), row),
            pl.BlockSpec((_BLK, d_dense), row),
            pl.BlockSpec((8, _N2), cst),
            pl.BlockSpec((d_dense, d_out), cst),
            pl.BlockSpec((1, d_out), cst),
        ],
        out_specs=[
            pl.BlockSpec((_BLK, d_out), row),
            pl.BlockSpec((_BLK, 1), row),
            pl.BlockSpec((_BLK, 1), row),
            pl.BlockSpec((_BLK, 1), row),
            pl.BlockSpec((_BLK, 1), row),
            pl.BlockSpec((_BLK, 1), row),
            pl.BlockSpec((_BLK, 1), row),
        ],
        out_shape=[
            jax.ShapeDtypeStruct((_N1, d_out), jnp.float32),
            jax.ShapeDtypeStruct((_N1, 1), jnp.int32),
            jax.ShapeDtypeStruct((_N1, 1), jnp.int32),
            jax.ShapeDtypeStruct((_N1, 1), jnp.int32),
            jax.ShapeDtypeStruct((_N1, 1), jnp.float32),
            jax.ShapeDtypeStruct((_N1, 1), jnp.float32),
            jax.ShapeDtypeStruct((_N1, 1), jnp.float32),
        ],
    )(p1e, x1, p2e, w1, b2)

    sc_gather = functools.partial(
        pl.kernel,
        mesh=plsc.VectorSubcoreMesh(core_axis_name="c", subcore_axis_name="s"),
        out_type=[
            jax.ShapeDtypeStruct((_N1, d_sparse), jnp.float32),
            jax.ShapeDtypeStruct((_N1, d_sparse), jnp.float32),
            jax.ShapeDtypeStruct((_N1, d_sparse), jnp.float32),
        ],
        scratch_types=[
            pltpu.VMEM((_CHUNK,), jnp.int32),
            pltpu.VMEM((_CHUNK,), jnp.int32),
            pltpu.VMEM((_CHUNK,), jnp.int32),
            pltpu.VMEM((_CHUNK, d_sparse), jnp.float32),
            pltpu.VMEM((_CHUNK, d_sparse), jnp.float32),
            pltpu.VMEM((_CHUNK, d_sparse), jnp.float32),
            pltpu.SemaphoreType.DMA,
        ],
    )(_sc_gather_body)
    g0, g1, g2 = sc_gather(x2, i0.reshape(_N1), i1.reshape(_N1),
                           i2.reshape(_N1))

    first = lambda i: (jnp.where(i < _NB, i, 0), 0)
    second = lambda i: (jnp.where(i < _NB, 0, i - _NB), 0)
    x = pl.pallas_call(
        _bn_fused_body,
        grid=(2 * _NB,),
        in_specs=[
            pl.BlockSpec((_BLK, d_out), first),
            pl.BlockSpec((_BLK, d_sparse), first),
            pl.BlockSpec((_BLK, d_sparse), first),
            pl.BlockSpec((_BLK, d_sparse), first),
            pl.BlockSpec((_BLK, 1), first),
            pl.BlockSpec((_BLK, 1), first),
            pl.BlockSpec((_BLK, 1), first),
            pl.BlockSpec((d_sparse, d_out), cst),
            pl.BlockSpec((1, d_out), cst),
            pl.BlockSpec((1, d_out), cst),
        ],
        out_specs=pl.BlockSpec((_BLK, d_out), second),
        out_shape=jax.ShapeDtypeStruct((_N1, d_out), jnp.float32),
        scratch_shapes=[
            pltpu.VMEM((_N1, d_out), jnp.float32),
            pltpu.VMEM((1, d_out), jnp.float32),
            pltpu.VMEM((1, d_out), jnp.float32),
        ],
    )(h1, g0, g1, g2, w0, w1n, w2n, wm, gamma.reshape(1, d_out),
      beta.reshape(1, d_out))

    return (p1, x, o1)


# speedup vs baseline: 1.1727x; 1.0299x over previous
import functools

import jax
import jax.numpy as jnp
from jax import lax
from jax.experimental import pallas as pl
from jax.experimental.pallas import tpu as pltpu
from jax.experimental.pallas import tpu_sc as plsc

_N1, _N2 = 16384, 4096
_BLK = 512
_NB = _N1 // _BLK
_NW = 32
_CHUNK = 64
_NCH = (_N1 // _NW) // _CHUNK


def _knn_body(p1e_ref, p2e_ref,
              i0_ref, i1_ref, i2_ref, w0_ref, w1o_ref, w2o_ref):
    p1b = p1e_ref[...]
    p2t = p2e_ref[...]
    dot = jnp.dot(p1b, p2t, preferred_element_type=jnp.float32)
    n1 = jnp.sum(p1b * p1b, axis=1, keepdims=True)
    n2 = jnp.sum(p2t * p2t, axis=0, keepdims=True)
    d2 = (n1 + n2) - 2.0 * dot

    colsf = lax.broadcasted_iota(jnp.int32, d2.shape, 1).astype(jnp.float32)
    invs, sels = [], []
    for _ in range(3):
        m = jnp.min(d2, axis=1, keepdims=True)
        sel = jnp.min(jnp.where(d2 == m, colsf, float(_N2)),
                      axis=1, keepdims=True)
        dist = jnp.sqrt(jnp.maximum(m, 0.0))
        invs.append(1.0 / (dist + 1e-8))
        sels.append(sel)
        d2 = jnp.where(colsf == sel, jnp.inf, d2)
    wsum = invs[0] + invs[1] + invs[2]

    i0_ref[...] = sels[0].astype(jnp.int32)
    i1_ref[...] = sels[1].astype(jnp.int32)
    i2_ref[...] = sels[2].astype(jnp.int32)
    w0_ref[...] = invs[0] / wsum
    w1o_ref[...] = invs[1] / wsum
    w2o_ref[...] = invs[2] / wsum


def _sc_gather_body(x2_hbm, i0_hbm, i1_hbm, i2_hbm,
                    g0_hbm, g1_hbm, g2_hbm, iv, buf, gsem, osem):
    wid = lax.axis_index("s") * 2 + lax.axis_index("c")
    base = wid * (_N1 // _NW)
    idx_hbms = (i0_hbm, i1_hbm, i2_hbm)
    g_hbms = (g0_hbm, g1_hbm, g2_hbm)

    def load_idx(c):
        sl = pl.ds(base + c * _CHUNK, _CHUNK)
        for k in range(3):
            pltpu.sync_copy(idx_hbms[k].at[sl], iv.at[c % 2, k])

    def fire_gathers(c):
        return [pltpu.async_copy(x2_hbm.at[iv.at[c % 2, k]],
                                 buf.at[c % 2, k], gsem)
                for k in range(3)]

    def fire_out(c):
        sl = pl.ds(base + c * _CHUNK, _CHUNK)
        return [pltpu.async_copy(buf.at[c % 2, k], g_hbms[k].at[sl], osem)
                for k in range(3)]

    load_idx(0)
    gathers = fire_gathers(0)
    outs = []
    for c in range(_NCH):
        if c + 1 < _NCH:
            load_idx(c + 1)
        for g in gathers:
            g.wait()
        if c + 1 < _NCH:
            if c >= 1:
                for o in outs[c - 1]:
                    o.wait()
            gathers = fire_gathers(c + 1)
        outs.append(fire_out(c))
    for o in outs[_NCH - 2]:
        o.wait()
    for o in outs[_NCH - 1]:
        o.wait()


def _bn_fused_body(x1_ref, g0_ref, g1_ref, g2_ref, w0_ref, w1_ref, w2_ref,
                   wd_ref, wm_ref, b_ref, gamma_ref, beta_ref, x_ref,
                   h_scr, sum_scr, sumsq_scr):
    i = pl.program_id(0)

    @pl.when(i == 0)
    def _init():
        sum_scr[...] = jnp.zeros_like(sum_scr)
        sumsq_scr[...] = jnp.zeros_like(sumsq_scr)

    @pl.when(i < _NB)
    def _phase0():
        interp = (w0_ref[...] * g0_ref[...] + w1_ref[...] * g1_ref[...]
                  + w2_ref[...] * g2_ref[...])
        h = (jnp.dot(x1_ref[...], wd_ref[...],
                     preferred_element_type=jnp.float32)
             + jnp.dot(interp, wm_ref[...],
                       preferred_element_type=jnp.float32)
             + b_ref[...])
        h_scr[pl.ds(i * _BLK, _BLK), :] = h
        sum_scr[...] += jnp.sum(h, axis=0, keepdims=True)
        sumsq_scr[...] += jnp.sum(h * h, axis=0, keepdims=True)

    @pl.when(i >= _NB)
    def _phase1():
        j = i - _NB
        inv_n = 1.0 / _N1
        mean = sum_scr[...] * inv_n
        var = sumsq_scr[...] * inv_n - mean * mean
        scale = gamma_ref[...] / jnp.sqrt(var + 1e-5)
        shift = beta_ref[...] - mean * scale
        h = h_scr[pl.ds(j * _BLK, _BLK), :]
        x_ref[...] = jnp.maximum(h * scale + shift, 0.0)


def kernel(p1, x1, o1, p2, x2, o2, W, b, gamma, beta):
    d_dense = x1.shape[1]
    d_sparse = x2.shape[1]
    d_out = W.shape[1]

    p1e = jnp.pad(p1, ((0, 0), (0, 5)))
    p2e = jnp.pad(p2, ((0, 0), (0, 5))).T

    wd = W[:d_dense]
    wm = W[d_dense:]

    cst = lambda i: (0, 0)
    row = lambda i: (i, 0)
    i0, i1, i2, w0, w1n, w2n = pl.pallas_call(
        _knn_body,
        grid=(_NB,),
        in_specs=[
            pl.BlockSpec((_BLK, 8), row),
            pl.BlockSpec((8, _N2), cst),
        ],
        out_specs=[pl.BlockSpec((_BLK, 1), row)] * 6,
        out_shape=[jax.ShapeDtypeStruct((_N1, 1), jnp.int32)] * 3
                  + [jax.ShapeDtypeStruct((_N1, 1), jnp.float32)] * 3,
    )(p1e, p2e)

    sc_gather = functools.partial(
        pl.kernel,
        mesh=plsc.VectorSubcoreMesh(core_axis_name="c", subcore_axis_name="s"),
        out_type=[
            jax.ShapeDtypeStruct((_N1, d_sparse), jnp.float32),
            jax.ShapeDtypeStruct((_N1, d_sparse), jnp.float32),
            jax.ShapeDtypeStruct((_N1, d_sparse), jnp.float32),
        ],
        scratch_types=[
            pltpu.VMEM((2, 3, _CHUNK), jnp.int32),
            pltpu.VMEM((2, 3, _CHUNK, d_sparse), jnp.float32),
            pltpu.SemaphoreType.DMA,
            pltpu.SemaphoreType.DMA,
        ],
    )(_sc_gather_body)
    g0, g1, g2 = sc_gather(x2, i0.reshape(_N1), i1.reshape(_N1),
                           i2.reshape(_N1))

    first = lambda i: (jnp.where(i < _NB, i, 0), 0)
    second = lambda i: (jnp.where(i < _NB, 0, i - _NB), 0)
    x = pl.pallas_call(
        _bn_fused_body,
        grid=(2 * _NB,),
        in_specs=[
            pl.BlockSpec((_BLK, d_dense), first),
            pl.BlockSpec((_BLK, d_sparse), first),
            pl.BlockSpec((_BLK, d_sparse), first),
            pl.BlockSpec((_BLK, d_sparse), first),
            pl.BlockSpec((_BLK, 1), first),
            pl.BlockSpec((_BLK, 1), first),
            pl.BlockSpec((_BLK, 1), first),
            pl.BlockSpec((d_dense, d_out), cst),
            pl.BlockSpec((d_sparse, d_out), cst),
            pl.BlockSpec((1, d_out), cst),
            pl.BlockSpec((1, d_out), cst),
            pl.BlockSpec((1, d_out), cst),
        ],
        out_specs=pl.BlockSpec((_BLK, d_out), second),
        out_shape=jax.ShapeDtypeStruct((_N1, d_out), jnp.float32),
        scratch_shapes=[
            pltpu.VMEM((_N1, d_out), jnp.float32),
            pltpu.VMEM((1, d_out), jnp.float32),
            pltpu.VMEM((1, d_out), jnp.float32),
        ],
    )(x1, g0, g1, g2, w0, w1n, w2n, wd, wm, b.reshape(1, d_out),
      gamma.reshape(1, d_out), beta.reshape(1, d_out))

    return (p1, x, o1)
